# Initial kernel scaffold; baseline (speedup 1.0000x reference)
import jax
import jax.numpy as jnp
from jax.experimental import pallas as pl


def kernel(X, row_pointers, column_index, blockPartition, edgeToColumn, edgeToRow, W):
    raise NotImplementedError("write your pallas kernel here")


# speedup vs baseline: 109.8179x; 109.8179x over previous
import functools

import jax
import jax.numpy as jnp
from jax import lax
from jax.experimental import pallas as pl
from jax.experimental.pallas import tpu as pltpu
from jax.experimental.pallas import tpu_sc as plsc

N = 10000
E = 320000
D = 128

NC = 2
NS = 16
L = 16
NW = NC * NS
EW = E // NW
K = 80
NCHUNK = EW // K
RP_PAD = 16384
N_PAD = 10240
RB = 128
RPT = N_PAD // NS


def _mm_body(x_ref, w_ref, o_ref):
    o_ref[...] = jnp.dot(x_ref[...], w_ref[...],
                         preferred_element_type=jnp.float32)


def _matmul(X, W):
    M, BM = X.shape[0], 400
    return pl.pallas_call(
        _mm_body,
        grid=(M // BM,),
        in_specs=[pl.BlockSpec((BM, D), lambda i: (i, 0)),
                  pl.BlockSpec((D, D), lambda i: (0, 0))],
        out_specs=pl.BlockSpec((BM, D), lambda i: (i, 0)),
        out_shape=jax.ShapeDtypeStruct((M, D), jnp.float32),
    )(X, W)


def _add_body(a_ref, b_ref, o_ref):
    o_ref[...] = a_ref[...] + b_ref[...]


def _combine(partials):
    BM = 400
    return pl.pallas_call(
        _add_body,
        grid=(N // BM,),
        in_specs=[pl.BlockSpec((BM, D), lambda i: (i, 0)),
                  pl.BlockSpec((BM, D), lambda i: (i, 0))],
        out_specs=pl.BlockSpec((BM, D), lambda i: (i, 0)),
        out_shape=jax.ShapeDtypeStruct((N, D), jnp.float32),
    )(partials[:N], partials[N_PAD:N_PAD + N])


def _sc_body(xp_hbm, col_hbm, rp_hbm, out_hbm,
             rp_v, idx_v, seg_v, rows_v, stage_v, part_sh, sem):
    c = lax.axis_index("c")
    s = lax.axis_index("s")
    wid = c * NS + s

    pltpu.sync_copy(rp_hbm, rp_v)

    def _zrow(i, carry):
        for j in range(D // L):
            stage_v[i, pl.ds(j * L, L)] = jnp.zeros((L,), jnp.float32)
        return carry
    lax.fori_loop(0, RB, _zrow, 0)
    for b in range(RPT // RB):
        pltpu.sync_copy(stage_v, part_sh.at[pl.ds(s * RPT + b * RB, RB)])
    plsc.subcore_barrier()

    base = wid * EW

    def _chunk(i, carry):
        off = base + i * K
        pltpu.sync_copy(col_hbm.at[pl.ds(off, K)], idx_v)
        cp = pltpu.async_copy(xp_hbm.at[idx_v], rows_v, sem)
        for v in range(K // L):
            evec = off + v * L + lax.broadcasted_iota(jnp.int32, (L,), 0)
            pos = jnp.zeros((L,), jnp.int32)
            bit = RP_PAD // 2
            while bit:
                cand = pos + bit
                val = plsc.load_gather(rp_v, [cand])
                pos = jnp.where(val <= evec, cand, pos)
                bit //= 2
            seg_v[pl.ds(v * L, L)] = pos
        cp.wait()
        pltpu.sync_copy(rows_v, part_sh.at[seg_v], add=True)
        return carry

    lax.fori_loop(0, NCHUNK, _chunk, 0)
    plsc.subcore_barrier()

    for b in range(RPT // RB):
        r0 = s * RPT + b * RB
        pltpu.sync_copy(part_sh.at[pl.ds(r0, RB)], stage_v)
        pltpu.sync_copy(stage_v, out_hbm.at[pl.ds(c * N_PAD + r0, RB)])


def _sc_spmm(Xp, column_index, rp_pad):
    mesh = plsc.VectorSubcoreMesh(core_axis_name="c", subcore_axis_name="s")
    k = pl.kernel(
        _sc_body,
        out_type=jax.ShapeDtypeStruct((NC * N_PAD, D), jnp.float32),
        mesh=mesh,
        scratch_types=[
            pltpu.VMEM((RP_PAD,), jnp.int32),
            pltpu.VMEM((K,), jnp.int32),
            pltpu.VMEM((K,), jnp.int32),
            pltpu.VMEM((K, D), jnp.float32),
            pltpu.VMEM((RB, D), jnp.float32),
            pltpu.VMEM_SHARED((N_PAD, D), jnp.float32),
            pltpu.SemaphoreType.DMA,
        ],
        compiler_params=pltpu.CompilerParams(needs_layout_passes=False),
    )
    return k(Xp, column_index, rp_pad)


def kernel(X, row_pointers, column_index, blockPartition, edgeToColumn,
           edgeToRow, W):
    rp_pad = jnp.full((RP_PAD,), E, dtype=jnp.int32)
    rp_pad = rp_pad.at[:N + 1].set(row_pointers)
    rp_pad = rp_pad.at[0].set(0)
    rp_pad = rp_pad.at[N].set(E)

    Xp = _matmul(X, W)
    partials = _sc_spmm(Xp, column_index, rp_pad)
    return _combine(partials)
